# Initial kernel scaffold; baseline (speedup 1.0000x reference)
import jax
import jax.numpy as jnp
from jax.experimental import pallas as pl


def kernel(user_emb, entity_emb, edge_index, edge_type, inter_edge, inter_edge_w, relation_emb, W_Q, W_UI):
    raise NotImplementedError("write your pallas kernel here")


# speedup vs baseline: 7.0133x; 7.0133x over previous
import functools

import jax
import jax.numpy as jnp
from jax import lax
from jax.experimental import pallas as pl
from jax.experimental.pallas import tpu as pltpu
from jax.experimental.pallas import tpu_sc as plsc

N_ENT = 50000
N_USR = 50000
E_KG = 800000
E_UI = 400000
D = 64
H = 32
NR = 16
N_HOPS = 2

RB = 5000
NB = N_ENT // RB

NC = 2
NS = 16
NW = NC * NS
CH = 128

ROWS_PER_TILE = N_ENT // NS
ZROWS = 125


def _prep_body(e_ref, u_ref, rel_ref, wq_ref, wui_ref,
               pq_ref, pie_ref, pu_ref, t_ref, ec_ref):
    h = pl.program_id(0)
    r = pl.program_id(2)
    e_blk = e_ref[...]
    rel = rel_ref[...]
    rel_r = rel_ref[pl.ds(r, 1), :]
    e_lo = e_blk[:, :H]
    e_hi = e_blk[:, H:]
    eh = jnp.where(h == 0, e_lo, e_hi)
    relh = jnp.where(h == 0, rel_r[:, :H], rel_r[:, H:])
    t_ref[...] = eh * relh
    ec_ref[...] = eh

    @pl.when(r == 0)
    def _():
        u_blk = u_ref[...]
        aq = jnp.dot(e_blk, wq_ref[...], preferred_element_type=jnp.float32)
        ai = jnp.dot(e_blk, wui_ref[...], preferred_element_type=jnp.float32)
        au = jnp.dot(u_blk, wui_ref[...], preferred_element_type=jnp.float32)
        dn = (((1,), (1,)), ((), ()))
        pq_ref[...] = lax.dot_general(aq, rel, dn,
                                      preferred_element_type=jnp.float32)
        pie_ref[...] = lax.dot_general(ai, rel, dn,
                                       preferred_element_type=jnp.float32)
        pu_ref[...] = lax.dot_general(au, rel, dn,
                                      preferred_element_type=jnp.float32)


def _tc_prep(e, u, rel, wq, wui):
    f32 = jnp.float32
    return pl.pallas_call(
        _prep_body,
        grid=(2, NB, NR),
        in_specs=[
            pl.BlockSpec((RB, D), lambda h, b, r: (b, 0)),
            pl.BlockSpec((RB, D), lambda h, b, r: (b, 0)),
            pl.BlockSpec((NR, D), lambda h, b, r: (0, 0)),
            pl.BlockSpec((D, D), lambda h, b, r: (0, 0)),
            pl.BlockSpec((D, D), lambda h, b, r: (0, 0)),
        ],
        out_specs=[
            pl.BlockSpec((RB, NR), lambda h, b, r: (b, 0)),
            pl.BlockSpec((RB, NR), lambda h, b, r: (b, 0)),
            pl.BlockSpec((RB, NR), lambda h, b, r: (b, 0)),
            pl.BlockSpec((RB, H), lambda h, b, r: (h * NR * NB + r * NB + b, 0)),
            pl.BlockSpec((RB, H), lambda h, b, r: (h * NB + b, 0)),
        ],
        out_shape=[
            jax.ShapeDtypeStruct((N_ENT, NR), f32),
            jax.ShapeDtypeStruct((N_ENT, NR), f32),
            jax.ShapeDtypeStruct((N_USR, NR), f32),
            jax.ShapeDtypeStruct((2 * NR * N_ENT, H), f32),
            jax.ShapeDtypeStruct((2 * N_ENT, H), f32),
        ],
    )(e, u, rel, wq, wui)


def _fin_body(ae_ref, au_ref, rese_ref, resu_ref,
              e_ref, u_ref, rese_o_ref, resu_o_ref):
    def norm_cat(a_ref):
        lo = a_ref[0]
        hi = a_ref[1]
        sq = jnp.sum(lo * lo + hi * hi, axis=1, keepdims=True)
        inv = 1.0 / jnp.maximum(jnp.sqrt(sq), 1e-12)
        return jnp.concatenate([lo * inv, hi * inv], axis=1)

    e_new = norm_cat(ae_ref)
    u_new = norm_cat(au_ref)
    e_ref[...] = e_new
    u_ref[...] = u_new
    rese_o_ref[...] = rese_ref[...] + e_new
    resu_o_ref[...] = resu_ref[...] + u_new


def _tc_fin(agg_e, agg_u, res_e, res_u):
    f32 = jnp.float32
    return pl.pallas_call(
        _fin_body,
        grid=(NB,),
        in_specs=[
            pl.BlockSpec((2, RB, H), lambda b: (0, b, 0)),
            pl.BlockSpec((2, RB, H), lambda b: (0, b, 0)),
            pl.BlockSpec((RB, D), lambda b: (b, 0)),
            pl.BlockSpec((RB, D), lambda b: (b, 0)),
        ],
        out_specs=[
            pl.BlockSpec((RB, D), lambda b: (b, 0)),
            pl.BlockSpec((RB, D), lambda b: (b, 0)),
            pl.BlockSpec((RB, D), lambda b: (b, 0)),
            pl.BlockSpec((RB, D), lambda b: (b, 0)),
        ],
        out_shape=[
            jax.ShapeDtypeStruct((N_ENT, D), f32),
            jax.ShapeDtypeStruct((N_USR, D), f32),
            jax.ShapeDtypeStruct((N_ENT, D), f32),
            jax.ShapeDtypeStruct((N_USR, D), f32),
        ],
    )(agg_e, agg_u, res_e, res_u)


_KG_CHUNKS = E_KG // CH
_UI_CHUNKS = E_UI // CH
_I32 = jnp.int32


def _lanes():
    return lax.iota(_I32, 16)


_GDN = lax.GatherDimensionNumbers(
    offset_dims=(), collapsed_slice_dims=(0,), start_index_map=(0,))


def _bcast_lane(v, lane):
    idx = jnp.full((16,), lane, _I32)
    return lax.gather(v, idx[:, None], _GDN, slice_sizes=(1,),
                      mode=lax.GatherScatterMode.PROMISE_IN_BOUNDS)


def _scores_body(pq_hbm, pie_hbm, pu_hbm, head_hbm, tail_hbm, et_hbm,
                 uix_hbm, iix_hbm, iew_hbm,
                 wkg_hbm, gidx_hbm, wui_hbm,
                 hi_v, ti_v, rt_v, ph_v, pt_v, wv_v, gi_v, ew_v, last_v, sem):
    c = lax.axis_index("c")
    s = lax.axis_index("s")
    wid = s * NC + c
    lanes = _lanes()

    pltpu.sync_copy(et_hbm.at[pl.ds(E_KG - 16, 16)], last_v)
    r_last = _bcast_lane(last_v[...], 15)
    colstar = (r_last + 15) & 15

    @pl.loop(wid, _KG_CHUNKS, step=NW)
    def _kg(k):
        base = k * CH
        pltpu.sync_copy(head_hbm.at[pl.ds(base, CH)], hi_v)
        pltpu.sync_copy(tail_hbm.at[pl.ds(base, CH)], ti_v)
        pltpu.sync_copy(et_hbm.at[pl.ds(base, CH)], rt_v)
        pltpu.async_copy(pq_hbm.at[hi_v], ph_v, sem).wait()
        pltpu.async_copy(pq_hbm.at[ti_v], pt_v, sem).wait()
        for j in range(CH // 16):
            rows = lanes + j * 16
            r = rt_v[pl.ds(j * 16, 16)]
            col = (r + 15) & 15
            s1 = plsc.load_gather(ph_v, [rows, col])
            s2 = plsc.load_gather(pt_v, [rows, col])
            wv_v[pl.ds(j * 16, 16)] = jnp.exp(s1 + s2)
            gi_v[pl.ds(j * 16, 16)] = col * N_ENT + ti_v[pl.ds(j * 16, 16)]
        pltpu.sync_copy(wv_v, wkg_hbm.at[pl.ds(base, CH)])
        pltpu.sync_copy(gi_v, gidx_hbm.at[pl.ds(base, CH)])

    @pl.loop(wid, _UI_CHUNKS, step=NW)
    def _ui(k):
        base = k * CH
        pltpu.sync_copy(uix_hbm.at[pl.ds(base, CH)], hi_v)
        pltpu.sync_copy(iix_hbm.at[pl.ds(base, CH)], ti_v)
        pltpu.sync_copy(iew_hbm.at[pl.ds(base, CH)], ew_v)
        pltpu.async_copy(pu_hbm.at[hi_v], ph_v, sem).wait()
        pltpu.async_copy(pie_hbm.at[ti_v], pt_v, sem).wait()
        for j in range(CH // 16):
            rows = lanes + j * 16
            s1 = plsc.load_gather(ph_v, [rows, colstar])
            s2 = plsc.load_gather(pt_v, [rows, colstar])
            ew = ew_v[pl.ds(j * 16, 16)]
            wv_v[pl.ds(j * 16, 16)] = jnp.exp(s1 + s2) * ew
        pltpu.sync_copy(wv_v, wui_hbm.at[pl.ds(base, CH)])


def _sc_scores(pq, pie, pu, head, tail, et, uix, iix, iew):
    f32 = jnp.float32
    mesh = plsc.VectorSubcoreMesh(core_axis_name="c", subcore_axis_name="s")
    kfn = pl.kernel(
        _scores_body,
        out_type=[
            jax.ShapeDtypeStruct((E_KG,), f32),
            jax.ShapeDtypeStruct((E_KG,), _I32),
            jax.ShapeDtypeStruct((E_UI,), f32),
        ],
        mesh=mesh,
        scratch_types=[
            pltpu.VMEM((CH,), _I32),
            pltpu.VMEM((CH,), _I32),
            pltpu.VMEM((CH,), _I32),
            pltpu.VMEM((CH, NR), f32),
            pltpu.VMEM((CH, NR), f32),
            pltpu.VMEM((CH,), f32),
            pltpu.VMEM((CH,), _I32),
            pltpu.VMEM((CH,), f32),
            pltpu.VMEM((16,), _I32),
            pltpu.SemaphoreType.DMA,
        ],
        compiler_params=pltpu.CompilerParams(needs_layout_passes=False, use_tc_tiling_on_sc=False),
    )
    return kfn(pq, pie, pu, head, tail, et, uix, iix, iew)


def _agg_body(n_edges, half_rows, n_dst,
              tab_hbm, gix_hbm, six_hbm, w_hbm,
              out_hbm,
              gi_v, si_v, w_v, rows_v, zb_v, acc_sh, sem):
    c = lax.axis_index("c")
    s = lax.axis_index("s")
    half_base = c * half_rows
    rows_per_tile = n_dst // NS
    tile_row0 = s * rows_per_tile

    @pl.loop(0, ZROWS)
    def _z(i):
        zb_v[i, pl.ds(0, 16)] = jnp.zeros((16,), jnp.float32)
        zb_v[i, pl.ds(16, 16)] = jnp.zeros((16,), jnp.float32)

    @pl.loop(0, rows_per_tile // ZROWS)
    def _zc(q):
        pltpu.sync_copy(zb_v, acc_sh.at[pl.ds(tile_row0 + q * ZROWS, ZROWS)])

    plsc.subcore_barrier()

    n_chunks = n_edges // CH

    @pl.loop(s, n_chunks, step=NS)
    def _chunk(k):
        base = k * CH
        pltpu.sync_copy(gix_hbm.at[pl.ds(base, CH)], gi_v)
        pltpu.sync_copy(six_hbm.at[pl.ds(base, CH)], si_v)
        pltpu.sync_copy(w_hbm.at[pl.ds(base, CH)], w_v)
        for j in range(CH // 16):
            gi_v[pl.ds(j * 16, 16)] = gi_v[pl.ds(j * 16, 16)] + half_base
        pltpu.async_copy(tab_hbm.at[gi_v], rows_v, sem).wait()

        @pl.loop(0, CH // 16)
        def _scale(g):
            w16 = w_v[pl.ds(g * 16, 16)]

            @pl.loop(0, 16)
            def _lane(l):
                wb = _bcast_lane(w16, l)
                e = g * 16 + l
                r0 = rows_v[e, pl.ds(0, 16)]
                rows_v[e, pl.ds(0, 16)] = r0 * wb
                r1 = rows_v[e, pl.ds(16, 16)]
                rows_v[e, pl.ds(16, 16)] = r1 * wb

        pltpu.sync_copy(rows_v, acc_sh.at[si_v], add=True)

    plsc.subcore_barrier()

    @pl.loop(0, rows_per_tile // ZROWS)
    def _out(q):
        r0 = tile_row0 + q * ZROWS
        pltpu.sync_copy(acc_sh.at[pl.ds(r0, ZROWS)],
                        out_hbm.at[c, pl.ds(r0, ZROWS)])


def _sc_agg(tab, gix, six, w, n_edges, half_rows, n_dst):
    f32 = jnp.float32
    mesh = plsc.VectorSubcoreMesh(core_axis_name="c", subcore_axis_name="s")
    body = functools.partial(_agg_body, n_edges, half_rows, n_dst)
    kfn = pl.kernel(
        body,
        out_type=jax.ShapeDtypeStruct((2, n_dst, H), f32),
        mesh=mesh,
        scratch_types=[
            pltpu.VMEM((CH,), _I32),
            pltpu.VMEM((CH,), _I32),
            pltpu.VMEM((CH,), f32),
            pltpu.VMEM((CH, H), f32),
            pltpu.VMEM((ZROWS, H), f32),
            pltpu.VMEM_SHARED((n_dst, H), f32),
            pltpu.SemaphoreType.DMA,
        ],
        compiler_params=pltpu.CompilerParams(needs_layout_passes=False, use_tc_tiling_on_sc=False),
    )
    return kfn(tab, gix, six, w)


def kernel(user_emb, entity_emb, edge_index, edge_type, inter_edge,
           inter_edge_w, relation_emb, W_Q, W_UI):
    head = edge_index[0]
    tail = edge_index[1]
    uix = inter_edge[0]
    iix = inter_edge[1]

    e = entity_emb
    u = user_emb
    res_e = entity_emb
    res_u = user_emb

    for _ in range(N_HOPS):
        pq, pie, pu, t_cat, e_cat = _tc_prep(e, u, relation_emb, W_Q, W_UI)
        wkg, gidx, wui = _sc_scores(pq, pie, pu, head, tail, edge_type,
                                    uix, iix, inter_edge_w)
        agg_e = _sc_agg(t_cat, gidx, head, wkg, E_KG, NR * N_ENT, N_ENT)
        agg_u = _sc_agg(e_cat, iix, uix, wui, E_UI, N_ENT, N_USR)
        e, u, res_e, res_u = _tc_fin(agg_e, agg_u, res_e, res_u)

    return (res_e, res_u)


# speedup vs baseline: 13.6380x; 1.9446x over previous
import functools

import jax
import jax.numpy as jnp
from jax import lax
from jax.experimental import pallas as pl
from jax.experimental.pallas import tpu as pltpu
from jax.experimental.pallas import tpu_sc as plsc

N_ENT = 50000
N_USR = 50000
E_KG = 800000
E_UI = 400000
D = 64
H = 32
NR = 16
N_HOPS = 2

RB = 5000
NB = N_ENT // RB

NC = 2
NS = 16
NW = NC * NS
CH = 128

ROWS_PER_TILE = N_ENT // NS
ZROWS = 125


def _prep_body(e_ref, u_ref, rel_ref, wq_ref, wui_ref,
               pq_ref, pie_ref, pu_ref, t_ref, ec_ref):
    h = pl.program_id(0)
    r = pl.program_id(2)
    e_blk = e_ref[...]
    rel = rel_ref[...]
    rel_r = rel_ref[pl.ds(r, 1), :]
    e_lo = e_blk[:, :H]
    e_hi = e_blk[:, H:]
    eh = jnp.where(h == 0, e_lo, e_hi)
    relh = jnp.where(h == 0, rel_r[:, :H], rel_r[:, H:])
    t_ref[...] = eh * relh
    ec_ref[...] = eh

    @pl.when(r == 0)
    def _():
        u_blk = u_ref[...]
        aq = jnp.dot(e_blk, wq_ref[...], preferred_element_type=jnp.float32)
        ai = jnp.dot(e_blk, wui_ref[...], preferred_element_type=jnp.float32)
        au = jnp.dot(u_blk, wui_ref[...], preferred_element_type=jnp.float32)
        dn = (((1,), (1,)), ((), ()))
        pq_ref[...] = lax.dot_general(aq, rel, dn,
                                      preferred_element_type=jnp.float32)
        pie_ref[...] = lax.dot_general(ai, rel, dn,
                                       preferred_element_type=jnp.float32)
        pu_ref[...] = lax.dot_general(au, rel, dn,
                                      preferred_element_type=jnp.float32)


def _tc_prep(e, u, rel, wq, wui):
    f32 = jnp.float32
    return pl.pallas_call(
        _prep_body,
        grid=(2, NB, NR),
        in_specs=[
            pl.BlockSpec((RB, D), lambda h, b, r: (b, 0)),
            pl.BlockSpec((RB, D), lambda h, b, r: (b, 0)),
            pl.BlockSpec((NR, D), lambda h, b, r: (0, 0)),
            pl.BlockSpec((D, D), lambda h, b, r: (0, 0)),
            pl.BlockSpec((D, D), lambda h, b, r: (0, 0)),
        ],
        out_specs=[
            pl.BlockSpec((RB, NR), lambda h, b, r: (b, 0)),
            pl.BlockSpec((RB, NR), lambda h, b, r: (b, 0)),
            pl.BlockSpec((RB, NR), lambda h, b, r: (b, 0)),
            pl.BlockSpec((RB, H), lambda h, b, r: (h * NR * NB + r * NB + b, 0)),
            pl.BlockSpec((RB, H), lambda h, b, r: (h * NB + b, 0)),
        ],
        out_shape=[
            jax.ShapeDtypeStruct((N_ENT, NR), f32),
            jax.ShapeDtypeStruct((N_ENT, NR), f32),
            jax.ShapeDtypeStruct((N_USR, NR), f32),
            jax.ShapeDtypeStruct((2 * NR * N_ENT, H), f32),
            jax.ShapeDtypeStruct((2 * N_ENT, H), f32),
        ],
    )(e, u, rel, wq, wui)


def _fin_body(ae_ref, au_ref, rese_ref, resu_ref,
              e_ref, u_ref, rese_o_ref, resu_o_ref):
    def norm_cat(a_ref):
        lo = a_ref[0]
        hi = a_ref[1]
        sq = jnp.sum(lo * lo + hi * hi, axis=1, keepdims=True)
        inv = 1.0 / jnp.maximum(jnp.sqrt(sq), 1e-12)
        return jnp.concatenate([lo * inv, hi * inv], axis=1)

    e_new = norm_cat(ae_ref)
    u_new = norm_cat(au_ref)
    e_ref[...] = e_new
    u_ref[...] = u_new
    rese_o_ref[...] = rese_ref[...] + e_new
    resu_o_ref[...] = resu_ref[...] + u_new


def _tc_fin(agg_e, agg_u, res_e, res_u):
    f32 = jnp.float32
    return pl.pallas_call(
        _fin_body,
        grid=(NB,),
        in_specs=[
            pl.BlockSpec((2, RB, H), lambda b: (0, b, 0)),
            pl.BlockSpec((2, RB, H), lambda b: (0, b, 0)),
            pl.BlockSpec((RB, D), lambda b: (b, 0)),
            pl.BlockSpec((RB, D), lambda b: (b, 0)),
        ],
        out_specs=[
            pl.BlockSpec((RB, D), lambda b: (b, 0)),
            pl.BlockSpec((RB, D), lambda b: (b, 0)),
            pl.BlockSpec((RB, D), lambda b: (b, 0)),
            pl.BlockSpec((RB, D), lambda b: (b, 0)),
        ],
        out_shape=[
            jax.ShapeDtypeStruct((N_ENT, D), f32),
            jax.ShapeDtypeStruct((N_USR, D), f32),
            jax.ShapeDtypeStruct((N_ENT, D), f32),
            jax.ShapeDtypeStruct((N_USR, D), f32),
        ],
    )(agg_e, agg_u, res_e, res_u)


_KG_CHUNKS = E_KG // CH
_UI_CHUNKS = E_UI // CH
_I32 = jnp.int32

SS = 5
SE = SS * CH
_KG_SCH = _KG_CHUNKS // SS
_UI_SCH = _UI_CHUNKS // SS

CW = 64
AGG_SE = SS * CW


def _lanes():
    return lax.iota(_I32, 16)


_GDN = lax.GatherDimensionNumbers(
    offset_dims=(), collapsed_slice_dims=(0,), start_index_map=(0,))


def _bcast_lane(v, lane):
    idx = jnp.full((16,), lane, _I32)
    return lax.gather(v, idx[:, None], _GDN, slice_sizes=(1,),
                      mode=lax.GatherScatterMode.PROMISE_IN_BOUNDS)


def _scores_body(pq_hbm, pie_hbm, pu_hbm, head_hbm, tail_hbm, et_hbm,
                 uix_hbm, iix_hbm, iew_hbm,
                 wkg_hbm, gidx_hbm, wui_hbm,
                 hi_v, ti_v, rt_v, ph_v, pt_v, wv_v, gi_v, ew_v, last_v,
                 gsem, psem):
    c = lax.axis_index("c")
    s = lax.axis_index("s")
    wid = s * NC + c
    lanes = _lanes()

    pltpu.sync_copy(et_hbm.at[_KG_CHUNKS - 1, pl.ds(CH - 16, 16)], last_v)
    r_last = _bcast_lane(last_v[...], 15)
    colstar = (r_last + 15) & 15

    def kg_prep(k, b):
        row0 = k * SS
        d1 = pltpu.async_copy(head_hbm.at[pl.ds(row0, SS)], hi_v.at[b], psem)
        d2 = pltpu.async_copy(tail_hbm.at[pl.ds(row0, SS)], ti_v.at[b], psem)
        d3 = pltpu.async_copy(et_hbm.at[pl.ds(row0, SS)], rt_v.at[b], psem)
        d1.wait(); d2.wait(); d3.wait()
        dd = []
        for j in range(SS):
            dd.append(pltpu.async_copy(pq_hbm.at[hi_v.at[b, j]],
                                       ph_v.at[b, j], gsem))
            dd.append(pltpu.async_copy(pq_hbm.at[ti_v.at[b, j]],
                                       pt_v.at[b, j], gsem))
        return dd

    def kg_proc(k, b):
        row0 = k * SS
        for j in range(SS):
            @pl.loop(0, CH // 16)
            def _g(g):
                rows = lanes + g * 16
                r = rt_v[b, j, pl.ds(g * 16, 16)]
                col = (r + 15) & 15
                s1 = plsc.load_gather(ph_v.at[b, j], [rows, col])
                s2 = plsc.load_gather(pt_v.at[b, j], [rows, col])
                wv_v[b, j, pl.ds(g * 16, 16)] = jnp.exp(s1 + s2)
                gi_v[b, j, pl.ds(g * 16, 16)] = (
                    col * N_ENT + ti_v[b, j, pl.ds(g * 16, 16)])
        pltpu.sync_copy(wv_v.at[b], wkg_hbm.at[pl.ds(row0, SS)])
        pltpu.sync_copy(gi_v.at[b], gidx_hbm.at[pl.ds(row0, SS)])

    def kg_drain(b):
        for j in range(SS):
            pltpu.make_async_copy(pq_hbm.at[hi_v.at[b, j]],
                                  ph_v.at[b, j], gsem).wait()
            pltpu.make_async_copy(pq_hbm.at[ti_v.at[b, j]],
                                  pt_v.at[b, j], gsem).wait()

    @pl.loop(wid, _KG_SCH, step=2 * NW)
    def _kg(k):
        dd0 = kg_prep(k, 0)
        k1 = k + NW

        @pl.when(k1 < _KG_SCH)
        def _():
            kg_prep(k1, 1)

        for d in dd0:
            d.wait()
        kg_proc(k, 0)

        @pl.when(k1 < _KG_SCH)
        def _():
            kg_drain(1)
            kg_proc(k1, 1)

    def ui_prep(k, b):
        row0 = k * SS
        d1 = pltpu.async_copy(uix_hbm.at[pl.ds(row0, SS)], hi_v.at[b], psem)
        d2 = pltpu.async_copy(iix_hbm.at[pl.ds(row0, SS)], ti_v.at[b], psem)
        d3 = pltpu.async_copy(iew_hbm.at[pl.ds(row0, SS)], ew_v.at[b], psem)
        d1.wait(); d2.wait(); d3.wait()
        dd = []
        for j in range(SS):
            dd.append(pltpu.async_copy(pu_hbm.at[hi_v.at[b, j]],
                                       ph_v.at[b, j], gsem))
            dd.append(pltpu.async_copy(pie_hbm.at[ti_v.at[b, j]],
                                       pt_v.at[b, j], gsem))
        return dd

    def ui_proc(k, b):
        row0 = k * SS
        for j in range(SS):
            @pl.loop(0, CH // 16)
            def _g(g):
                rows = lanes + g * 16
                s1 = plsc.load_gather(ph_v.at[b, j], [rows, colstar])
                s2 = plsc.load_gather(pt_v.at[b, j], [rows, colstar])
                ew = ew_v[b, j, pl.ds(g * 16, 16)]
                wv_v[b, j, pl.ds(g * 16, 16)] = jnp.exp(s1 + s2) * ew
        pltpu.sync_copy(wv_v.at[b], wui_hbm.at[pl.ds(row0, SS)])

    def ui_drain(b):
        for j in range(SS):
            pltpu.make_async_copy(pu_hbm.at[hi_v.at[b, j]],
                                  ph_v.at[b, j], gsem).wait()
            pltpu.make_async_copy(pie_hbm.at[ti_v.at[b, j]],
                                  pt_v.at[b, j], gsem).wait()

    @pl.loop(wid, _UI_SCH, step=2 * NW)
    def _ui(k):
        dd0 = ui_prep(k, 0)
        k1 = k + NW

        @pl.when(k1 < _UI_SCH)
        def _():
            ui_prep(k1, 1)

        for d in dd0:
            d.wait()
        ui_proc(k, 0)

        @pl.when(k1 < _UI_SCH)
        def _():
            ui_drain(1)
            ui_proc(k1, 1)


def _sc_scores(pq, pie, pu, head, tail, et, uix, iix, iew):
    f32 = jnp.float32
    mesh = plsc.VectorSubcoreMesh(core_axis_name="c", subcore_axis_name="s")
    kfn = pl.kernel(
        _scores_body,
        out_type=[
            jax.ShapeDtypeStruct((_KG_CHUNKS, CH), f32),
            jax.ShapeDtypeStruct((_KG_CHUNKS, CH), _I32),
            jax.ShapeDtypeStruct((_UI_CHUNKS, CH), f32),
        ],
        mesh=mesh,
        scratch_types=[
            pltpu.VMEM((2, SS, CH), _I32),
            pltpu.VMEM((2, SS, CH), _I32),
            pltpu.VMEM((2, SS, CH), _I32),
            pltpu.VMEM((2, SS, CH, NR), f32),
            pltpu.VMEM((2, SS, CH, NR), f32),
            pltpu.VMEM((2, SS, CH), f32),
            pltpu.VMEM((2, SS, CH), _I32),
            pltpu.VMEM((2, SS, CH), f32),
            pltpu.VMEM((16,), _I32),
            pltpu.SemaphoreType.DMA,
            pltpu.SemaphoreType.DMA,
        ],
        compiler_params=pltpu.CompilerParams(needs_layout_passes=False, use_tc_tiling_on_sc=False),
    )
    return kfn(pq, pie, pu, head, tail, et, uix, iix, iew)


def _agg_body(n_sch, half_rows, n_dst,
              tab_hbm, gix_hbm, six_hbm, w_hbm,
              out_hbm,
              gi_v, si_v, w_v, rows_v, zb_v, acc_sh, gsem, psem):
    c = lax.axis_index("c")
    s = lax.axis_index("s")
    half_base = c * half_rows
    rows_per_tile = n_dst // NS
    tile_row0 = s * rows_per_tile

    @pl.loop(0, ZROWS)
    def _z(i):
        zb_v[i, pl.ds(0, 16)] = jnp.zeros((16,), jnp.float32)
        zb_v[i, pl.ds(16, 16)] = jnp.zeros((16,), jnp.float32)

    @pl.loop(0, rows_per_tile // ZROWS)
    def _zc(q):
        pltpu.sync_copy(zb_v, acc_sh.at[pl.ds(tile_row0 + q * ZROWS, ZROWS)])

    plsc.subcore_barrier()

    def prep(k, b):
        row0 = k * SS
        d1 = pltpu.async_copy(gix_hbm.at[pl.ds(row0, SS)], gi_v.at[b], psem)
        d2 = pltpu.async_copy(six_hbm.at[pl.ds(row0, SS)], si_v.at[b], psem)
        d3 = pltpu.async_copy(w_hbm.at[pl.ds(row0, SS)], w_v.at[b], psem)
        d1.wait(); d2.wait(); d3.wait()
        for j in range(SS):
            for l in range(CW // 16):
                gi_v[b, j, pl.ds(l * 16, 16)] = (
                    gi_v[b, j, pl.ds(l * 16, 16)] + half_base)
        dd = []
        for j in range(SS):
            dd.append(pltpu.async_copy(tab_hbm.at[gi_v.at[b, j]],
                                       rows_v.at[b, j], gsem))
        return dd

    def drain(b):
        for j in range(SS):
            pltpu.make_async_copy(tab_hbm.at[gi_v.at[b, j]],
                                  rows_v.at[b, j], gsem).wait()

    def proc(b):
        for j in range(SS):
            @pl.loop(0, CW // 16)
            def _g(g):
                w16 = w_v[b, j, pl.ds(g * 16, 16)]
                for l in range(16):
                    wb = _bcast_lane(w16, l)
                    e = g * 16 + l
                    r0 = rows_v[b, j, e, pl.ds(0, 16)]
                    rows_v[b, j, e, pl.ds(0, 16)] = r0 * wb
                    r1 = rows_v[b, j, e, pl.ds(16, 16)]
                    rows_v[b, j, e, pl.ds(16, 16)] = r1 * wb
        for j in range(SS):
            pltpu.sync_copy(rows_v.at[b, j], acc_sh.at[si_v.at[b, j]],
                            add=True)

    @pl.loop(s, n_sch, step=2 * NS)
    def _sch(k):
        dd0 = prep(k, 0)
        k1 = k + NS

        @pl.when(k1 < n_sch)
        def _():
            prep(k1, 1)

        for d in dd0:
            d.wait()
        proc(0)

        @pl.when(k1 < n_sch)
        def _():
            drain(1)
            proc(1)

    plsc.subcore_barrier()

    @pl.loop(0, rows_per_tile // ZROWS)
    def _out(q):
        r0 = tile_row0 + q * ZROWS
        pltpu.sync_copy(acc_sh.at[pl.ds(r0, ZROWS)],
                        out_hbm.at[c, pl.ds(r0, ZROWS)])


def _sc_agg(tab, gix, six, w, n_edges, half_rows, n_dst):
    f32 = jnp.float32
    mesh = plsc.VectorSubcoreMesh(core_axis_name="c", subcore_axis_name="s")
    body = functools.partial(_agg_body, n_edges // AGG_SE, half_rows, n_dst)
    kfn = pl.kernel(
        body,
        out_type=jax.ShapeDtypeStruct((2, n_dst, H), f32),
        mesh=mesh,
        scratch_types=[
            pltpu.VMEM((2, SS, CW), _I32),
            pltpu.VMEM((2, SS, CW), _I32),
            pltpu.VMEM((2, SS, CW), f32),
            pltpu.VMEM((2, SS, CW, H), f32),
            pltpu.VMEM((ZROWS, H), f32),
            pltpu.VMEM_SHARED((n_dst, H), f32),
            pltpu.SemaphoreType.DMA,
            pltpu.SemaphoreType.DMA,
        ],
        compiler_params=pltpu.CompilerParams(needs_layout_passes=False, use_tc_tiling_on_sc=False),
    )
    return kfn(tab, gix, six, w)


def kernel(user_emb, entity_emb, edge_index, edge_type, inter_edge,
           inter_edge_w, relation_emb, W_Q, W_UI):
    head = edge_index[0].reshape(_KG_CHUNKS, CH)
    tail = edge_index[1].reshape(_KG_CHUNKS, CH)
    et = edge_type.reshape(_KG_CHUNKS, CH)
    uix = inter_edge[0].reshape(_UI_CHUNKS, CH)
    iix = inter_edge[1].reshape(_UI_CHUNKS, CH)
    iew = inter_edge_w.reshape(_UI_CHUNKS, CH)

    e = entity_emb
    u = user_emb
    res_e = entity_emb
    res_u = user_emb

    head_w = head.reshape(E_KG // CW, CW)
    uix_w = uix.reshape(E_UI // CW, CW)
    iix_w = iix.reshape(E_UI // CW, CW)

    for _ in range(N_HOPS):
        pq, pie, pu, t_cat, e_cat = _tc_prep(e, u, relation_emb, W_Q, W_UI)
        wkg, gidx, wui = _sc_scores(pq, pie, pu, head, tail, et,
                                    uix, iix, iew)
        agg_e = _sc_agg(t_cat, gidx.reshape(E_KG // CW, CW), head_w,
                        wkg.reshape(E_KG // CW, CW),
                        E_KG, NR * N_ENT, N_ENT)
        agg_u = _sc_agg(e_cat, iix_w, uix_w,
                        wui.reshape(E_UI // CW, CW),
                        E_UI, N_ENT, N_USR)
        e, u, res_e, res_u = _tc_fin(agg_e, agg_u, res_e, res_u)

    return (res_e, res_u)


# speedup vs baseline: 14.4411x; 1.0589x over previous
import functools

import jax
import jax.numpy as jnp
from jax import lax
from jax.experimental import pallas as pl
from jax.experimental.pallas import tpu as pltpu
from jax.experimental.pallas import tpu_sc as plsc

N_ENT = 50000
N_USR = 50000
E_KG = 800000
E_UI = 400000
D = 64
H = 32
NR = 16
N_HOPS = 2

RB = 5000
NB = N_ENT // RB

NC = 2
NS = 16
NW = NC * NS
CH = 128

ROWS_PER_TILE = N_ENT // NS
ZROWS = 125


def _prep_body(e_ref, u_ref, rel_ref, wq_ref, wui_ref,
               pq_ref, pie_ref, pu_ref, t_ref, ec_ref):
    h = pl.program_id(0)
    r = pl.program_id(2)
    e_blk = e_ref[...]
    rel = rel_ref[...]
    rel_r = rel_ref[pl.ds(r, 1), :]
    e_lo = e_blk[:, :H]
    e_hi = e_blk[:, H:]
    eh = jnp.where(h == 0, e_lo, e_hi)
    relh = jnp.where(h == 0, rel_r[:, :H], rel_r[:, H:])
    t_ref[...] = eh * relh
    ec_ref[...] = eh

    @pl.when(r == 0)
    def _():
        u_blk = u_ref[...]
        aq = jnp.dot(e_blk, wq_ref[...], preferred_element_type=jnp.float32)
        ai = jnp.dot(e_blk, wui_ref[...], preferred_element_type=jnp.float32)
        au = jnp.dot(u_blk, wui_ref[...], preferred_element_type=jnp.float32)
        dn = (((1,), (1,)), ((), ()))
        pq_ref[...] = lax.dot_general(aq, rel, dn,
                                      preferred_element_type=jnp.float32)
        pie_ref[...] = lax.dot_general(ai, rel, dn,
                                       preferred_element_type=jnp.float32)
        pu_ref[...] = lax.dot_general(au, rel, dn,
                                      preferred_element_type=jnp.float32)


def _tc_prep(e, u, rel, wq, wui):
    f32 = jnp.float32
    return pl.pallas_call(
        _prep_body,
        grid=(2, NB, NR),
        in_specs=[
            pl.BlockSpec((RB, D), lambda h, b, r: (b, 0)),
            pl.BlockSpec((RB, D), lambda h, b, r: (b, 0)),
            pl.BlockSpec((NR, D), lambda h, b, r: (0, 0)),
            pl.BlockSpec((D, D), lambda h, b, r: (0, 0)),
            pl.BlockSpec((D, D), lambda h, b, r: (0, 0)),
        ],
        out_specs=[
            pl.BlockSpec((RB, NR), lambda h, b, r: (b, 0)),
            pl.BlockSpec((RB, NR), lambda h, b, r: (b, 0)),
            pl.BlockSpec((RB, NR), lambda h, b, r: (b, 0)),
            pl.BlockSpec((RB, H), lambda h, b, r: (h * NR * NB + r * NB + b, 0)),
            pl.BlockSpec((RB, H), lambda h, b, r: (h * NB + b, 0)),
        ],
        out_shape=[
            jax.ShapeDtypeStruct((N_ENT, NR), f32),
            jax.ShapeDtypeStruct((N_ENT, NR), f32),
            jax.ShapeDtypeStruct((N_USR, NR), f32),
            jax.ShapeDtypeStruct((2 * NR * N_ENT, H), f32),
            jax.ShapeDtypeStruct((2 * N_ENT, H), f32),
        ],
    )(e, u, rel, wq, wui)


def _fin_body(ae_ref, au_ref, rese_ref, resu_ref,
              e_ref, u_ref, rese_o_ref, resu_o_ref):
    def norm_cat(a_ref):
        lo = a_ref[0]
        hi = a_ref[1]
        sq = jnp.sum(lo * lo + hi * hi, axis=1, keepdims=True)
        inv = 1.0 / jnp.maximum(jnp.sqrt(sq), 1e-12)
        return jnp.concatenate([lo * inv, hi * inv], axis=1)

    e_new = norm_cat(ae_ref)
    u_new = norm_cat(au_ref)
    e_ref[...] = e_new
    u_ref[...] = u_new
    rese_o_ref[...] = rese_ref[...] + e_new
    resu_o_ref[...] = resu_ref[...] + u_new


def _tc_fin(agg_e, agg_u, res_e, res_u):
    f32 = jnp.float32
    return pl.pallas_call(
        _fin_body,
        grid=(NB,),
        in_specs=[
            pl.BlockSpec((2, RB, H), lambda b: (0, b, 0)),
            pl.BlockSpec((2, RB, H), lambda b: (0, b, 0)),
            pl.BlockSpec((RB, D), lambda b: (b, 0)),
            pl.BlockSpec((RB, D), lambda b: (b, 0)),
        ],
        out_specs=[
            pl.BlockSpec((RB, D), lambda b: (b, 0)),
            pl.BlockSpec((RB, D), lambda b: (b, 0)),
            pl.BlockSpec((RB, D), lambda b: (b, 0)),
            pl.BlockSpec((RB, D), lambda b: (b, 0)),
        ],
        out_shape=[
            jax.ShapeDtypeStruct((N_ENT, D), f32),
            jax.ShapeDtypeStruct((N_USR, D), f32),
            jax.ShapeDtypeStruct((N_ENT, D), f32),
            jax.ShapeDtypeStruct((N_USR, D), f32),
        ],
    )(agg_e, agg_u, res_e, res_u)


_KG_CHUNKS = E_KG // CH
_UI_CHUNKS = E_UI // CH
_I32 = jnp.int32

SS = 5
SE = SS * CH
_KG_SCH = _KG_CHUNKS // SS
_UI_SCH = _UI_CHUNKS // SS

CW = 64
AGG_SE = SS * CW


def _lanes():
    return lax.iota(_I32, 16)


_GDN = lax.GatherDimensionNumbers(
    offset_dims=(), collapsed_slice_dims=(0,), start_index_map=(0,))


def _bcast_lane(v, lane):
    idx = jnp.full((16,), lane, _I32)
    return lax.gather(v, idx[:, None], _GDN, slice_sizes=(1,),
                      mode=lax.GatherScatterMode.PROMISE_IN_BOUNDS)


def _scores_body(pq_hbm, pie_hbm, pu_hbm, head_hbm, tail_hbm, et_hbm,
                 uix_hbm, iix_hbm, iew_hbm,
                 wkg_hbm, gidx_hbm, wui_hbm,
                 hi_v, ti_v, rt_v, s1_v, s2_v, wv_v, gi_v, ew_v, last_v,
                 gsem, psem):
    c = lax.axis_index("c")
    s = lax.axis_index("s")
    wid = s * NC + c

    pltpu.sync_copy(et_hbm.at[_KG_CHUNKS - 1, pl.ds(CH - 16, 16)], last_v)
    r_last = _bcast_lane(last_v[...], 15)
    colstar = (r_last + 15) & 15

    def kg_prep(k, b):
        row0 = k * SS
        d1 = pltpu.async_copy(head_hbm.at[pl.ds(row0, SS)], hi_v.at[b], psem)
        d2 = pltpu.async_copy(tail_hbm.at[pl.ds(row0, SS)], ti_v.at[b], psem)
        d3 = pltpu.async_copy(et_hbm.at[pl.ds(row0, SS)], rt_v.at[b], psem)
        d1.wait(); d2.wait(); d3.wait()
        for j in range(SS):
            @pl.loop(0, CH // 16)
            def _g(g):
                sl = pl.ds(g * 16, 16)
                r = rt_v[b, j, sl]
                col = (r + 15) & 15
                t16 = ti_v[b, j, sl]
                gi_v[b, j, sl] = col * N_ENT + t16
                hi_v[b, j, sl] = hi_v[b, j, sl] * NR + col
                ti_v[b, j, sl] = t16 * NR + col
        dd = []
        for j in range(SS):
            dd.append(pltpu.async_copy(pq_hbm.at[hi_v.at[b, j]],
                                       s1_v.at[b, j], gsem))
            dd.append(pltpu.async_copy(pq_hbm.at[ti_v.at[b, j]],
                                       s2_v.at[b, j], gsem))
        return dd

    def kg_proc(k, b):
        row0 = k * SS
        for j in range(SS):
            @pl.loop(0, CH // 16)
            def _g(g):
                sl = pl.ds(g * 16, 16)
                wv_v[b, j, sl] = jnp.exp(s1_v[b, j, sl] + s2_v[b, j, sl])
        pltpu.sync_copy(wv_v.at[b], wkg_hbm.at[pl.ds(row0, SS)])
        pltpu.sync_copy(gi_v.at[b], gidx_hbm.at[pl.ds(row0, SS)])

    def kg_drain(b):
        for j in range(SS):
            pltpu.make_async_copy(pq_hbm.at[hi_v.at[b, j]],
                                  s1_v.at[b, j], gsem).wait()
            pltpu.make_async_copy(pq_hbm.at[ti_v.at[b, j]],
                                  s2_v.at[b, j], gsem).wait()

    @pl.loop(wid, _KG_SCH, step=2 * NW)
    def _kg(k):
        dd0 = kg_prep(k, 0)
        k1 = k + NW

        @pl.when(k1 < _KG_SCH)
        def _():
            kg_prep(k1, 1)

        for d in dd0:
            d.wait()
        kg_proc(k, 0)

        @pl.when(k1 < _KG_SCH)
        def _():
            kg_drain(1)
            kg_proc(k1, 1)

    def ui_prep(k, b):
        row0 = k * SS
        d1 = pltpu.async_copy(uix_hbm.at[pl.ds(row0, SS)], hi_v.at[b], psem)
        d2 = pltpu.async_copy(iix_hbm.at[pl.ds(row0, SS)], ti_v.at[b], psem)
        d3 = pltpu.async_copy(iew_hbm.at[pl.ds(row0, SS)], ew_v.at[b], psem)
        d1.wait(); d2.wait(); d3.wait()
        for j in range(SS):
            @pl.loop(0, CH // 16)
            def _g(g):
                sl = pl.ds(g * 16, 16)
                hi_v[b, j, sl] = hi_v[b, j, sl] * NR + colstar
                ti_v[b, j, sl] = ti_v[b, j, sl] * NR + colstar
        dd = []
        for j in range(SS):
            dd.append(pltpu.async_copy(pu_hbm.at[hi_v.at[b, j]],
                                       s1_v.at[b, j], gsem))
            dd.append(pltpu.async_copy(pie_hbm.at[ti_v.at[b, j]],
                                       s2_v.at[b, j], gsem))
        return dd

    def ui_proc(k, b):
        row0 = k * SS
        for j in range(SS):
            @pl.loop(0, CH // 16)
            def _g(g):
                sl = pl.ds(g * 16, 16)
                wv_v[b, j, sl] = (jnp.exp(s1_v[b, j, sl] + s2_v[b, j, sl])
                                  * ew_v[b, j, sl])
        pltpu.sync_copy(wv_v.at[b], wui_hbm.at[pl.ds(row0, SS)])

    def ui_drain(b):
        for j in range(SS):
            pltpu.make_async_copy(pu_hbm.at[hi_v.at[b, j]],
                                  s1_v.at[b, j], gsem).wait()
            pltpu.make_async_copy(pie_hbm.at[ti_v.at[b, j]],
                                  s2_v.at[b, j], gsem).wait()

    @pl.loop(wid, _UI_SCH, step=2 * NW)
    def _ui(k):
        dd0 = ui_prep(k, 0)
        k1 = k + NW

        @pl.when(k1 < _UI_SCH)
        def _():
            ui_prep(k1, 1)

        for d in dd0:
            d.wait()
        ui_proc(k, 0)

        @pl.when(k1 < _UI_SCH)
        def _():
            ui_drain(1)
            ui_proc(k1, 1)


def _sc_scores(pq, pie, pu, head, tail, et, uix, iix, iew):
    f32 = jnp.float32
    mesh = plsc.VectorSubcoreMesh(core_axis_name="c", subcore_axis_name="s")
    kfn = pl.kernel(
        _scores_body,
        out_type=[
            jax.ShapeDtypeStruct((_KG_CHUNKS, CH), f32),
            jax.ShapeDtypeStruct((_KG_CHUNKS, CH), _I32),
            jax.ShapeDtypeStruct((_UI_CHUNKS, CH), f32),
        ],
        mesh=mesh,
        scratch_types=[
            pltpu.VMEM((2, SS, CH), _I32),
            pltpu.VMEM((2, SS, CH), _I32),
            pltpu.VMEM((2, SS, CH), _I32),
            pltpu.VMEM((2, SS, CH), f32),
            pltpu.VMEM((2, SS, CH), f32),
            pltpu.VMEM((2, SS, CH), f32),
            pltpu.VMEM((2, SS, CH), _I32),
            pltpu.VMEM((2, SS, CH), f32),
            pltpu.VMEM((16,), _I32),
            pltpu.SemaphoreType.DMA,
            pltpu.SemaphoreType.DMA,
        ],
        compiler_params=pltpu.CompilerParams(needs_layout_passes=False, use_tc_tiling_on_sc=False),
    )
    return kfn(pq.reshape(N_ENT * NR), pie.reshape(N_ENT * NR),
               pu.reshape(N_USR * NR), head, tail, et, uix, iix, iew)


def _agg_body(n_sch, half_rows, n_dst,
              tab_hbm, gix_hbm, six_hbm, w_hbm,
              out_hbm,
              gi_v, si_v, w_v, rows_v, zb_v, acc_sh, gsem, psem, ssem):
    c = lax.axis_index("c")
    s = lax.axis_index("s")
    half_base = c * half_rows
    rows_per_tile = n_dst // NS
    tile_row0 = s * rows_per_tile

    @pl.loop(0, ZROWS)
    def _z(i):
        zb_v[i, pl.ds(0, 16)] = jnp.zeros((16,), jnp.float32)
        zb_v[i, pl.ds(16, 16)] = jnp.zeros((16,), jnp.float32)

    @pl.loop(0, rows_per_tile // ZROWS)
    def _zc(q):
        pltpu.sync_copy(zb_v, acc_sh.at[pl.ds(tile_row0 + q * ZROWS, ZROWS)])

    plsc.subcore_barrier()

    def wait_scatter(b):
        for j in range(SS):
            pltpu.make_async_copy(rows_v.at[b, j],
                                  acc_sh.at[si_v.at[b, j]], ssem).wait()

    def fire_scatter(b):
        for j in range(SS):
            pltpu.async_copy(rows_v.at[b, j], acc_sh.at[si_v.at[b, j]],
                             ssem, add=True)

    zero16 = jnp.zeros((16,), jnp.float32)
    for b in range(2):
        for j in range(SS):
            @pl.loop(0, CW)
            def _zr(e):
                si_v[b, j, pl.ds((e // 16) * 16, 16)] = jnp.zeros((16,), _I32)
                rows_v[b, j, e, pl.ds(0, 16)] = zero16
                rows_v[b, j, e, pl.ds(16, 16)] = zero16
        fire_scatter(b)

    def prep(k, b):
        wait_scatter(b)
        row0 = k * SS
        d1 = pltpu.async_copy(gix_hbm.at[pl.ds(row0, SS)], gi_v.at[b], psem)
        d2 = pltpu.async_copy(six_hbm.at[pl.ds(row0, SS)], si_v.at[b], psem)
        d3 = pltpu.async_copy(w_hbm.at[pl.ds(row0, SS)], w_v.at[b], psem)
        d1.wait(); d2.wait(); d3.wait()
        for j in range(SS):
            for l in range(CW // 16):
                gi_v[b, j, pl.ds(l * 16, 16)] = (
                    gi_v[b, j, pl.ds(l * 16, 16)] + half_base)
        dd = []
        for j in range(SS):
            dd.append(pltpu.async_copy(tab_hbm.at[gi_v.at[b, j]],
                                       rows_v.at[b, j], gsem))
        return dd

    def drain(b):
        for j in range(SS):
            pltpu.make_async_copy(tab_hbm.at[gi_v.at[b, j]],
                                  rows_v.at[b, j], gsem).wait()

    def proc(b):
        for j in range(SS):
            @pl.loop(0, CW // 16)
            def _g(g):
                w16 = w_v[b, j, pl.ds(g * 16, 16)]
                for l in range(16):
                    wb = _bcast_lane(w16, l)
                    e = g * 16 + l
                    r0 = rows_v[b, j, e, pl.ds(0, 16)]
                    rows_v[b, j, e, pl.ds(0, 16)] = r0 * wb
                    r1 = rows_v[b, j, e, pl.ds(16, 16)]
                    rows_v[b, j, e, pl.ds(16, 16)] = r1 * wb
        fire_scatter(b)

    @pl.loop(s, n_sch, step=2 * NS)
    def _sch(k):
        dd0 = prep(k, 0)
        k1 = k + NS

        @pl.when(k1 < n_sch)
        def _():
            prep(k1, 1)

        for d in dd0:
            d.wait()
        proc(0)

        @pl.when(k1 < n_sch)
        def _():
            drain(1)
            proc(1)

    wait_scatter(0)
    wait_scatter(1)
    plsc.subcore_barrier()

    @pl.loop(0, rows_per_tile // ZROWS)
    def _out(q):
        r0 = tile_row0 + q * ZROWS
        pltpu.sync_copy(acc_sh.at[pl.ds(r0, ZROWS)],
                        out_hbm.at[c, pl.ds(r0, ZROWS)])


def _sc_agg(tab, gix, six, w, n_edges, half_rows, n_dst):
    f32 = jnp.float32
    mesh = plsc.VectorSubcoreMesh(core_axis_name="c", subcore_axis_name="s")
    body = functools.partial(_agg_body, n_edges // AGG_SE, half_rows, n_dst)
    kfn = pl.kernel(
        body,
        out_type=jax.ShapeDtypeStruct((2, n_dst, H), f32),
        mesh=mesh,
        scratch_types=[
            pltpu.VMEM((2, SS, CW), _I32),
            pltpu.VMEM((2, SS, CW), _I32),
            pltpu.VMEM((2, SS, CW), f32),
            pltpu.VMEM((2, SS, CW, H), f32),
            pltpu.VMEM((ZROWS, H), f32),
            pltpu.VMEM_SHARED((n_dst, H), f32),
            pltpu.SemaphoreType.DMA,
            pltpu.SemaphoreType.DMA,
            pltpu.SemaphoreType.DMA,
        ],
        compiler_params=pltpu.CompilerParams(needs_layout_passes=False, use_tc_tiling_on_sc=False),
    )
    return kfn(tab, gix, six, w)


def kernel(user_emb, entity_emb, edge_index, edge_type, inter_edge,
           inter_edge_w, relation_emb, W_Q, W_UI):
    head = edge_index[0].reshape(_KG_CHUNKS, CH)
    tail = edge_index[1].reshape(_KG_CHUNKS, CH)
    et = edge_type.reshape(_KG_CHUNKS, CH)
    uix = inter_edge[0].reshape(_UI_CHUNKS, CH)
    iix = inter_edge[1].reshape(_UI_CHUNKS, CH)
    iew = inter_edge_w.reshape(_UI_CHUNKS, CH)

    e = entity_emb
    u = user_emb
    res_e = entity_emb
    res_u = user_emb

    head_w = head.reshape(E_KG // CW, CW)
    uix_w = uix.reshape(E_UI // CW, CW)
    iix_w = iix.reshape(E_UI // CW, CW)

    for _ in range(N_HOPS):
        pq, pie, pu, t_cat, e_cat = _tc_prep(e, u, relation_emb, W_Q, W_UI)
        wkg, gidx, wui = _sc_scores(pq, pie, pu, head, tail, et,
                                    uix, iix, iew)
        agg_e = _sc_agg(t_cat, gidx.reshape(E_KG // CW, CW), head_w,
                        wkg.reshape(E_KG // CW, CW),
                        E_KG, NR * N_ENT, N_ENT)
        agg_u = _sc_agg(e_cat, iix_w, uix_w,
                        wui.reshape(E_UI // CW, CW),
                        E_UI, N_ENT, N_USR)
        e, u, res_e, res_u = _tc_fin(agg_e, agg_u, res_e, res_u)

    return (res_e, res_u)
